# baseline (speedup 1.0000x reference)
import functools

import jax
import jax.numpy as jnp
from jax import lax
from jax.experimental import pallas as pl
from jax.experimental.pallas import tpu as pltpu
from jax.experimental.pallas import tpu_sc as plsc

_NBINS = 256
_MID = 64
_ITERS = 8
_NC, _NS, _L = 2, 16, 16
_NW = _NC * _NS
_B = 16
_H = 512
_W = 512
_HW = _H * _W
_HALF = _HW // 2
_CHR = 64
_NCH = (_H // 2) // _CHR
_U = 8
_UH = 16

_mesh = plsc.VectorSubcoreMesh(
    core_axis_name="c", subcore_axis_name="s",
    num_cores=_NC, num_subcores=_NS)


def _sc_body(v_hbm, mm_hbm, hist_hbm, bufs, hvals, stage, stage2,
             shared, sem0, sem1):
  c = lax.axis_index("c")
  s = lax.axis_index("s")
  wid = c * _NS + s
  b = wid // 2
  row0 = (wid % 2) * (_H // 2)
  sems = (sem0, sem1)

  def src(k):
    return v_hbm.at[b, pl.ds(row0 + k * _CHR, _CHR), :]

  mns = list(jnp.full((_L,), jnp.inf, jnp.float32) for _ in range(_U))
  mxs = list(jnp.full((_L,), -jnp.inf, jnp.float32) for _ in range(_U))
  pend = pltpu.async_copy(src(0), bufs.at[0], sems[0])
  for k in range(_NCH):
    nxt = None
    if k + 1 < _NCH:
      nxt = pltpu.async_copy(src(k + 1), bufs.at[(k + 1) % 2],
                             sems[(k + 1) % 2])
    pend.wait()

    def step(i, carry2, _k=k):
      mns2, mxs2 = carry2
      new_mns, new_mxs = list(mns2), list(mxs2)
      row = i >> 1
      col = (i & 1) * (_W // 2)
      for u in range(_W // (2 * _L)):
        x = bufs[_k % 2, row, pl.ds(col + u * _L, _L)]
        new_mns[u % _U] = jnp.minimum(new_mns[u % _U], x)
        new_mxs[u % _U] = jnp.maximum(new_mxs[u % _U], x)
      return tuple(new_mns), tuple(new_mxs)

    mns, mxs = plsc.parallel_loop(
        0, 2 * _CHR, carry=(tuple(mns), tuple(mxs)))(step)
    pend = nxt
  own_mn = functools.reduce(jnp.minimum, mns)
  own_mx = functools.reduce(jnp.maximum, mxs)
  stage[pl.ds(0, _L)] = own_mn
  stage[pl.ds(_L, _L)] = own_mx
  pltpu.sync_copy(stage, mm_hbm.at[pl.ds(wid * 2 * _L, 2 * _L)])
  pltpu.sync_copy(stage, shared.at[s])
  plsc.subcore_barrier()
  pltpu.sync_copy(shared.at[s ^ 1], stage2)
  mnv = jnp.minimum(own_mn, stage2[pl.ds(0, _L)])
  mxv = jnp.maximum(own_mx, stage2[pl.ds(_L, _L)])
  mn_s = mnv[0]
  mx_s = mxv[0]
  for i in range(1, _L):
    mn_s = jnp.minimum(mn_s, mnv[i])
    mx_s = jnp.maximum(mx_s, mxv[i])
  mn = jnp.broadcast_to(mn_s, (_L,))
  mx = jnp.broadcast_to(mx_s, (_L,))
  rng = mx - mn
  safe = jnp.where(rng == 0.0, jnp.float32(1.0), rng)
  inv = jnp.float32(_NBINS) / safe

  zero = jnp.zeros((_L,), jnp.float32)
  for j in range(_NBINS // _L):
    hvals[pl.ds(j * _L, _L)] = zero

  ones = jnp.ones((_L,), jnp.float32)
  pend = pltpu.async_copy(src(0), bufs.at[0], sems[0])
  for k in range(_NCH):
    nxt = None
    if k + 1 < _NCH:
      nxt = pltpu.async_copy(src(k + 1), bufs.at[(k + 1) % 2],
                             sems[(k + 1) % 2])
    pend.wait()

    def step(i, _k=k):
      row = i >> 1
      col = (i & 1) * (_W // 2)
      for u in range(_W // (2 * _L)):
        x = bufs[_k % 2, row, pl.ds(col + u * _L, _L)]
        t = (x - mn) * inv
        idx = jnp.minimum(t, jnp.float32(_NBINS - 1)).astype(jnp.int32)
        plsc.addupdate_scatter(hvals, [idx], ones)

    plsc.parallel_loop(0, 2 * _CHR)(step)
    pend = nxt

  pltpu.sync_copy(hvals, hist_hbm.at[pl.ds(wid * _NBINS, _NBINS)])


_sc_call = pl.kernel(
    _sc_body,
    out_type=(jax.ShapeDtypeStruct((_NW * 2 * _L,), jnp.float32),
              jax.ShapeDtypeStruct((_NW * _NBINS,), jnp.float32)),
    mesh=_mesh,
    scratch_types=[pltpu.VMEM((2, _CHR, _W), jnp.float32),
                   pltpu.VMEM((_NBINS,), jnp.float32),
                   pltpu.VMEM((2 * _L,), jnp.float32),
                   pltpu.VMEM((2 * _L,), jnp.float32),
                   pltpu.VMEM_SHARED((_NS, 2 * _L), jnp.float32),
                   pltpu.SemaphoreType.DMA,
                   pltpu.SemaphoreType.DMA],
    compiler_params=pltpu.CompilerParams(needs_layout_passes=False,
                                         use_tc_tiling_on_sc=True),
)


def _lrelu(x):
  return jnp.where(x >= 0, x, 0.01 * x)


def _mlp_body(hist_ref, mm_ref, mu_ref,
              w1, b1, w2, b2, w3, b3, w4, b4, w5, b5, out_ref):
  h3 = hist_ref[...].reshape(_B, 2, _NBINS)
  counts = h3[:, 0, :] + h3[:, 1, :]
  h = counts * jnp.float32(1.0 / _HW)
  m4 = mm_ref[...].reshape(_B, 2, 2, _L)
  mn = jnp.min(jnp.minimum(m4[:, 0, 0, :], m4[:, 1, 0, :]), axis=1,
               keepdims=True)
  mx = jnp.max(jnp.maximum(m4[:, 0, 1, :], m4[:, 1, 1, :]), axis=1,
               keepdims=True)
  vec = jnp.concatenate([h, mn, mx, mu_ref[...]], axis=1)
  x = _lrelu(vec @ w1[...] + b1[...])
  x = _lrelu(x @ w2[...] + b2[...])
  x = _lrelu(jnp.concatenate([x, vec], axis=1) @ w3[...] + b3[...])
  x = _lrelu(x @ w4[...] + b4[...])
  out_ref[...] = _lrelu(x @ w5[...] + b5[...])


_CB = 8


def _curve_body(a_ref, v_ref, o_ref):
  g = pl.program_id(0)
  for j in range(_CB):
    x = v_ref[j]
    for i in range(_ITERS):
      a = a_ref[g * _CB + j, i]
      x = x * ((1.0 + a) - a * x)
    o_ref[j] = x


def kernel(V_chanel, mu, W1, b1, W2, b2, W3, b3, W4, b4, W5, b5):
  v3 = V_chanel.reshape(_B, _H, _W)
  mm, hist = _sc_call(v3)

  alphas = pl.pallas_call(
      _mlp_body,
      out_shape=jax.ShapeDtypeStruct((_B, _ITERS), jnp.float32),
  )(hist.reshape(_NW, _NBINS), mm.reshape(_NW * 2, _L),
    mu, W1, b1, W2, b2, W3, b3, W4, b4, W5, b5)

  out = pl.pallas_call(
      _curve_body,
      grid=(_B // _CB,),
      in_specs=[
          pl.BlockSpec((_B, _ITERS), lambda b: (0, 0),
                       memory_space=pltpu.SMEM),
          pl.BlockSpec((_CB, _H, _W), lambda b: (b, 0, 0)),
      ],
      out_specs=pl.BlockSpec((_CB, _H, _W), lambda b: (b, 0, 0)),
      out_shape=jax.ShapeDtypeStruct((_B, _H, _W), jnp.float32),
  )(alphas, v3)
  return out.reshape(V_chanel.shape)


# speedup vs baseline: 1.0290x; 1.0290x over previous
import functools

import jax
import jax.numpy as jnp
from jax import lax
from jax.experimental import pallas as pl
from jax.experimental.pallas import tpu as pltpu
from jax.experimental.pallas import tpu_sc as plsc

_NBINS = 256
_MID = 64
_ITERS = 8
_NC, _NS, _L = 2, 16, 16
_NW = _NC * _NS
_B = 16
_H = 512
_W = 512
_HW = _H * _W
_HALF = _HW // 2
_CHR = 64
_NCH = (_H // 2) // _CHR
_U = 8
_UH = 16

_mesh = plsc.VectorSubcoreMesh(
    core_axis_name="c", subcore_axis_name="s",
    num_cores=_NC, num_subcores=_NS)


def _sc_body(v_hbm, mm_hbm, hist_hbm, bufs, hvals, stage, stage2,
             shared, sem0, sem1):
  c = lax.axis_index("c")
  s = lax.axis_index("s")
  wid = c * _NS + s
  b = wid // 2
  row0 = (wid % 2) * (_H // 2)
  sems = (sem0, sem1)

  def src(k):
    return v_hbm.at[b, pl.ds(row0 + k * _CHR, _CHR), :]

  mns = list(jnp.full((_L,), jnp.inf, jnp.float32) for _ in range(_U))
  mxs = list(jnp.full((_L,), -jnp.inf, jnp.float32) for _ in range(_U))
  pend = pltpu.async_copy(src(0), bufs.at[0], sems[0])
  for k in range(_NCH):
    nxt = None
    if k + 1 < _NCH:
      nxt = pltpu.async_copy(src(k + 1), bufs.at[(k + 1) % 2],
                             sems[(k + 1) % 2])
    pend.wait()

    def step(i, carry2, _k=k):
      mns2, mxs2 = carry2
      new_mns, new_mxs = list(mns2), list(mxs2)
      row = i >> 1
      col = (i & 1) * (_W // 2)
      for u in range(_W // (2 * _L)):
        x = bufs[_k % 2, row, pl.ds(col + u * _L, _L)]
        new_mns[u % _U] = jnp.minimum(new_mns[u % _U], x)
        new_mxs[u % _U] = jnp.maximum(new_mxs[u % _U], x)
      return tuple(new_mns), tuple(new_mxs)

    mns, mxs = plsc.parallel_loop(
        0, 2 * _CHR, carry=(tuple(mns), tuple(mxs)))(step)
    pend = nxt
  own_mn = functools.reduce(jnp.minimum, mns)
  own_mx = functools.reduce(jnp.maximum, mxs)
  stage[pl.ds(0, _L)] = own_mn
  stage[pl.ds(_L, _L)] = own_mx
  pltpu.sync_copy(stage, mm_hbm.at[pl.ds(wid * 2 * _L, 2 * _L)])
  pltpu.sync_copy(stage, shared.at[s])
  plsc.subcore_barrier()
  pltpu.sync_copy(shared.at[s ^ 1], stage2)
  mnv = jnp.minimum(own_mn, stage2[pl.ds(0, _L)])
  mxv = jnp.maximum(own_mx, stage2[pl.ds(_L, _L)])
  mn_s = mnv[0]
  mx_s = mxv[0]
  for i in range(1, _L):
    mn_s = jnp.minimum(mn_s, mnv[i])
    mx_s = jnp.maximum(mx_s, mxv[i])
  mn = jnp.broadcast_to(mn_s, (_L,))
  mx = jnp.broadcast_to(mx_s, (_L,))
  rng = mx - mn
  safe = jnp.where(rng == 0.0, jnp.float32(1.0), rng)
  inv = jnp.float32(_NBINS) / safe

  zero = jnp.zeros((_L,), jnp.float32)
  for j in range(_NBINS // _L):
    hvals[pl.ds(j * _L, _L)] = zero

  ones = jnp.ones((_L,), jnp.float32)
  pend = pltpu.async_copy(src(0), bufs.at[0], sems[0])
  for k in range(_NCH):
    nxt = None
    if k + 1 < _NCH:
      nxt = pltpu.async_copy(src(k + 1), bufs.at[(k + 1) % 2],
                             sems[(k + 1) % 2])
    pend.wait()

    def step(i, _k=k):
      row = i >> 1
      col = (i & 1) * (_W // 2)
      for u in range(_W // (2 * _L)):
        x = bufs[_k % 2, row, pl.ds(col + u * _L, _L)]
        t = (x - mn) * inv
        idx = jnp.minimum(t, jnp.float32(_NBINS - 1)).astype(jnp.int32)
        plsc.addupdate_scatter(hvals, [idx], ones)

    plsc.parallel_loop(0, 2 * _CHR)(step)
    pend = nxt

  pltpu.sync_copy(hvals, hist_hbm.at[pl.ds(wid * _NBINS, _NBINS)])


_sc_call = pl.kernel(
    _sc_body,
    out_type=(jax.ShapeDtypeStruct((_NW * 2 * _L,), jnp.float32),
              jax.ShapeDtypeStruct((_NW * _NBINS,), jnp.float32)),
    mesh=_mesh,
    scratch_types=[pltpu.VMEM((2, _CHR, _W), jnp.float32),
                   pltpu.VMEM((_NBINS,), jnp.float32),
                   pltpu.VMEM((2 * _L,), jnp.float32),
                   pltpu.VMEM((2 * _L,), jnp.float32),
                   pltpu.VMEM_SHARED((_NS, 2 * _L), jnp.float32),
                   pltpu.SemaphoreType.DMA,
                   pltpu.SemaphoreType.DMA],
    compiler_params=pltpu.CompilerParams(needs_layout_passes=False,
                                         use_tc_tiling_on_sc=True),
)


def _lrelu(x):
  return jnp.where(x >= 0, x, 0.01 * x)


def _mlp_body(hist_ref, mm_ref, mu_ref,
              w1, b1, w2, b2, w3, b3, w4, b4, w5, b5, out_ref):
  h3 = hist_ref[...].reshape(_B, 2, _NBINS)
  counts = h3[:, 0, :] + h3[:, 1, :]
  h = counts * jnp.float32(1.0 / _HW)
  m4 = mm_ref[...].reshape(_B, 2, 2, _L)
  mn = jnp.min(jnp.minimum(m4[:, 0, 0, :], m4[:, 1, 0, :]), axis=1,
               keepdims=True)
  mx = jnp.max(jnp.maximum(m4[:, 0, 1, :], m4[:, 1, 1, :]), axis=1,
               keepdims=True)
  vec = jnp.concatenate([h, mn, mx, mu_ref[...]], axis=1)
  x = _lrelu(vec @ w1[...] + b1[...])
  x = _lrelu(x @ w2[...] + b2[...])
  x = _lrelu(jnp.concatenate([x, vec], axis=1) @ w3[...] + b3[...])
  x = _lrelu(x @ w4[...] + b4[...])
  out_ref[...] = _lrelu(x @ w5[...] + b5[...])


_CB = 4


def _curve_body(a_ref, v_ref, o_ref):
  g = pl.program_id(0)
  for j in range(_CB):
    x = v_ref[j]
    for i in range(_ITERS):
      a = a_ref[g * _CB + j, i]
      x = x * ((1.0 + a) - a * x)
    o_ref[j] = x


def kernel(V_chanel, mu, W1, b1, W2, b2, W3, b3, W4, b4, W5, b5):
  v3 = V_chanel.reshape(_B, _H, _W)
  mm, hist = _sc_call(v3)

  alphas = pl.pallas_call(
      _mlp_body,
      out_shape=jax.ShapeDtypeStruct((_B, _ITERS), jnp.float32),
  )(hist.reshape(_NW, _NBINS), mm.reshape(_NW * 2, _L),
    mu, W1, b1, W2, b2, W3, b3, W4, b4, W5, b5)

  out = pl.pallas_call(
      _curve_body,
      grid=(_B // _CB,),
      in_specs=[
          pl.BlockSpec((_B, _ITERS), lambda b: (0, 0),
                       memory_space=pltpu.SMEM),
          pl.BlockSpec((_CB, _H, _W), lambda b: (b, 0, 0)),
      ],
      out_specs=pl.BlockSpec((_CB, _H, _W), lambda b: (b, 0, 0)),
      out_shape=jax.ShapeDtypeStruct((_B, _H, _W), jnp.float32),
  )(alphas, v3)
  return out.reshape(V_chanel.shape)


# speedup vs baseline: 1.0348x; 1.0056x over previous
import functools

import jax
import jax.numpy as jnp
from jax import lax
from jax.experimental import pallas as pl
from jax.experimental.pallas import tpu as pltpu
from jax.experimental.pallas import tpu_sc as plsc

_NBINS = 256
_MID = 64
_ITERS = 8
_NC, _NS, _L = 2, 16, 16
_NW = _NC * _NS
_B = 16
_H = 512
_W = 512
_HW = _H * _W
_HALF = _HW // 2
_CHR = 64
_NCH = (_H // 2) // _CHR
_U = 8
_UH = 16

_mesh = plsc.VectorSubcoreMesh(
    core_axis_name="c", subcore_axis_name="s",
    num_cores=_NC, num_subcores=_NS)


def _sc_body(v_hbm, mm_hbm, hist_hbm, bufs, hvals, stage, stage2,
             shared, sem0, sem1):
  c = lax.axis_index("c")
  s = lax.axis_index("s")
  wid = c * _NS + s
  b = wid // 2
  row0 = (wid % 2) * (_H // 2)
  sems = (sem0, sem1)

  def src(k):
    return v_hbm.at[b, pl.ds(row0 + k * _CHR, _CHR), :]

  mns = list(jnp.full((_L,), jnp.inf, jnp.float32) for _ in range(_U))
  mxs = list(jnp.full((_L,), -jnp.inf, jnp.float32) for _ in range(_U))
  pend = pltpu.async_copy(src(0), bufs.at[0], sems[0])
  for k in range(_NCH):
    nxt = None
    if k + 1 < _NCH:
      nxt = pltpu.async_copy(src(k + 1), bufs.at[(k + 1) % 2],
                             sems[(k + 1) % 2])
    pend.wait()

    def step(i, carry2, _k=k):
      mns2, mxs2 = carry2
      new_mns, new_mxs = list(mns2), list(mxs2)
      row = i >> 1
      col = (i & 1) * (_W // 2)
      for u in range(_W // (2 * _L)):
        x = bufs[_k % 2, row, pl.ds(col + u * _L, _L)]
        new_mns[u % _U] = jnp.minimum(new_mns[u % _U], x)
        new_mxs[u % _U] = jnp.maximum(new_mxs[u % _U], x)
      return tuple(new_mns), tuple(new_mxs)

    mns, mxs = plsc.parallel_loop(
        0, 2 * _CHR, carry=(tuple(mns), tuple(mxs)))(step)
    pend = nxt
  own_mn = functools.reduce(jnp.minimum, mns)
  own_mx = functools.reduce(jnp.maximum, mxs)
  stage[pl.ds(0, _L)] = own_mn
  stage[pl.ds(_L, _L)] = own_mx
  pltpu.sync_copy(stage, mm_hbm.at[pl.ds(wid * 2 * _L, 2 * _L)])
  pltpu.sync_copy(stage, shared.at[s])
  plsc.subcore_barrier()
  pltpu.sync_copy(shared.at[s ^ 1], stage2)
  mnv = jnp.minimum(own_mn, stage2[pl.ds(0, _L)])
  mxv = jnp.maximum(own_mx, stage2[pl.ds(_L, _L)])
  mn_s = mnv[0]
  mx_s = mxv[0]
  for i in range(1, _L):
    mn_s = jnp.minimum(mn_s, mnv[i])
    mx_s = jnp.maximum(mx_s, mxv[i])
  mn = jnp.broadcast_to(mn_s, (_L,))
  mx = jnp.broadcast_to(mx_s, (_L,))
  rng = mx - mn
  safe = jnp.where(rng == 0.0, jnp.float32(1.0), rng)
  inv = jnp.float32(_NBINS * (1.0 - 2.0 ** -20)) / safe
  bias = jnp.broadcast_to(jnp.float32(2.0 ** 23 - 0.5), (_L,))

  zero = jnp.zeros((_L,), jnp.float32)
  for j in range(_NBINS // _L):
    hvals[pl.ds(j * _L, _L)] = zero

  ones = jnp.ones((_L,), jnp.float32)
  pend = pltpu.async_copy(src(0), bufs.at[0], sems[0])
  for k in range(_NCH):
    nxt = None
    if k + 1 < _NCH:
      nxt = pltpu.async_copy(src(k + 1), bufs.at[(k + 1) % 2],
                             sems[(k + 1) % 2])
    pend.wait()

    def step(i, _k=k):
      row = i >> 1
      col = (i & 1) * (_W // 2)
      for u in range(_W // (2 * _L)):
        x = bufs[_k % 2, row, pl.ds(col + u * _L, _L)]
        y = (x - mn) * inv + bias
        idx = plsc.bitcast(y, jnp.int32) & 0xFF
        plsc.addupdate_scatter(hvals, [idx], ones)

    plsc.parallel_loop(0, 2 * _CHR)(step)
    pend = nxt

  pltpu.sync_copy(hvals, hist_hbm.at[pl.ds(wid * _NBINS, _NBINS)])


_sc_call = pl.kernel(
    _sc_body,
    out_type=(jax.ShapeDtypeStruct((_NW * 2 * _L,), jnp.float32),
              jax.ShapeDtypeStruct((_NW * _NBINS,), jnp.float32)),
    mesh=_mesh,
    scratch_types=[pltpu.VMEM((2, _CHR, _W), jnp.float32),
                   pltpu.VMEM((_NBINS,), jnp.float32),
                   pltpu.VMEM((2 * _L,), jnp.float32),
                   pltpu.VMEM((2 * _L,), jnp.float32),
                   pltpu.VMEM_SHARED((_NS, 2 * _L), jnp.float32),
                   pltpu.SemaphoreType.DMA,
                   pltpu.SemaphoreType.DMA],
    compiler_params=pltpu.CompilerParams(needs_layout_passes=False,
                                         use_tc_tiling_on_sc=True),
)


def _lrelu(x):
  return jnp.where(x >= 0, x, 0.01 * x)


def _mlp_body(hist_ref, mm_ref, mu_ref,
              w1, b1, w2, b2, w3, b3, w4, b4, w5, b5, out_ref):
  h3 = hist_ref[...].reshape(_B, 2, _NBINS)
  counts = h3[:, 0, :] + h3[:, 1, :]
  h = counts * jnp.float32(1.0 / _HW)
  m4 = mm_ref[...].reshape(_B, 2, 2, _L)
  mn = jnp.min(jnp.minimum(m4[:, 0, 0, :], m4[:, 1, 0, :]), axis=1,
               keepdims=True)
  mx = jnp.max(jnp.maximum(m4[:, 0, 1, :], m4[:, 1, 1, :]), axis=1,
               keepdims=True)
  vec = jnp.concatenate([h, mn, mx, mu_ref[...]], axis=1)
  x = _lrelu(vec @ w1[...] + b1[...])
  x = _lrelu(x @ w2[...] + b2[...])
  x = _lrelu(jnp.concatenate([x, vec], axis=1) @ w3[...] + b3[...])
  x = _lrelu(x @ w4[...] + b4[...])
  out_ref[...] = _lrelu(x @ w5[...] + b5[...])


_CB = 4


def _curve_body(a_ref, v_ref, o_ref):
  g = pl.program_id(0)
  for j in range(_CB):
    x = v_ref[j]
    for i in range(_ITERS):
      a = a_ref[g * _CB + j, i]
      x = x * ((1.0 + a) - a * x)
    o_ref[j] = x


def kernel(V_chanel, mu, W1, b1, W2, b2, W3, b3, W4, b4, W5, b5):
  v3 = V_chanel.reshape(_B, _H, _W)
  mm, hist = _sc_call(v3)

  alphas = pl.pallas_call(
      _mlp_body,
      out_shape=jax.ShapeDtypeStruct((_B, _ITERS), jnp.float32),
  )(hist.reshape(_NW, _NBINS), mm.reshape(_NW * 2, _L),
    mu, W1, b1, W2, b2, W3, b3, W4, b4, W5, b5)

  out = pl.pallas_call(
      _curve_body,
      grid=(_B // _CB,),
      in_specs=[
          pl.BlockSpec((_B, _ITERS), lambda b: (0, 0),
                       memory_space=pltpu.SMEM),
          pl.BlockSpec((_CB, _H, _W), lambda b: (b, 0, 0)),
      ],
      out_specs=pl.BlockSpec((_CB, _H, _W), lambda b: (b, 0, 0)),
      out_shape=jax.ShapeDtypeStruct((_B, _H, _W), jnp.float32),
  )(alphas, v3)
  return out.reshape(V_chanel.shape)
